# Initial kernel scaffold; baseline (speedup 1.0000x reference)
import functools

import jax
import jax.numpy as jnp
from jax import lax
from jax.experimental import pallas as pl
from jax.experimental.pallas import tpu as pltpu
from jax.experimental.pallas import tpu_sc as plsc

N = 10000
E = 640000
NC = 2
NS = 16
LANES = 16
NW = NC * NS
EW = E // NW
CH = 80
NCH = EW // CH
RPS = N // NS

D1, DP1 = 50, 64
D2, DP2 = 100, 112


def _msg_body(DP, m_hbm, src_hbm, dst_hbm, dist_hbm, zeros_hbm, out_hbm,
              idx_s, idx_d, dist_v, rows_v, agg_sh, sem):
    cid = lax.axis_index("c")
    sid = lax.axis_index("s")
    wid = cid * NS + sid
    base = wid * EW
    r0 = sid * RPS
    pltpu.sync_copy(zeros_hbm.at[pl.ds(r0, RPS)], agg_sh.at[pl.ds(r0, RPS)])
    plsc.subcore_barrier()

    def chunk(c, carry):
        off = base + c * CH
        pltpu.sync_copy(src_hbm.at[pl.ds(off, CH)], idx_s)
        pltpu.sync_copy(dst_hbm.at[pl.ds(off, CH)], idx_d)
        pltpu.sync_copy(dist_hbm.at[pl.ds(off, CH)], dist_v)
        pltpu.async_copy(m_hbm.at[idx_s], rows_v, sem).wait()
        for g in range(CH // LANES):
            dvec = dist_v[pl.ds(g * LANES, LANES)]
            for l in range(LANES):
                dsp = jnp.take(dvec, jnp.full((LANES,), l, jnp.int32),
                               mode="promise_in_bounds")
                e = g * LANES + l
                for j in range(DP // LANES):
                    sl = pl.ds(j * LANES, LANES)
                    rows_v[e, sl] = rows_v[e, sl] * dsp
        pltpu.sync_copy(rows_v, agg_sh.at[idx_d], add=True)
        return carry

    lax.fori_loop(0, NCH, chunk, 0)
    plsc.subcore_barrier()
    pltpu.sync_copy(agg_sh.at[pl.ds(r0, RPS)],
                    out_hbm.at[cid, pl.ds(r0, RPS)])


@functools.lru_cache(maxsize=None)
def _make_msg(DP):
    mesh = plsc.VectorSubcoreMesh(core_axis_name="c", subcore_axis_name="s")
    return pl.kernel(
        functools.partial(_msg_body, DP),
        out_type=jax.ShapeDtypeStruct((NC, N, DP), jnp.float32),
        mesh=mesh,
        scratch_types=[
            pltpu.VMEM((CH,), jnp.int32),
            pltpu.VMEM((CH,), jnp.int32),
            pltpu.VMEM((CH,), jnp.float32),
            pltpu.VMEM((CH, DP), jnp.float32),
            pltpu.VMEM_SHARED((N, DP), jnp.float32),
            pltpu.SemaphoreType.DMA,
        ],
    )


EW2 = E // 16
ECH = 800


def _edge_body(tbl_hbm, src_hbm, dst_hbm, out_hbm, tb_v, sidx, didx, out_v):
    cid = lax.axis_index("c")
    sid = lax.axis_index("s")
    wid = cid * NS + sid
    comp = wid // 16
    part = wid % 16
    base = part * EW2
    pltpu.sync_copy(tbl_hbm.at[comp], tb_v.at[0])
    pltpu.sync_copy(tbl_hbm.at[comp + 2], tb_v.at[1])
    z16 = jnp.zeros((LANES,), jnp.int32)

    def chunk(c, carry):
        off = base + c * ECH
        pltpu.sync_copy(src_hbm.at[pl.ds(off, ECH)], sidx)
        pltpu.sync_copy(dst_hbm.at[pl.ds(off, ECH)], didx)
        for g in range(ECH // LANES):
            sl = pl.ds(g * LANES, LANES)
            sv = sidx[sl]
            dv = didx[sl]
            s = (plsc.load_gather(tb_v, [z16, sv])
                 + plsc.load_gather(tb_v, [z16 + 1, dv]))
            out_v[sl] = s
        pltpu.sync_copy(out_v, out_hbm.at[comp, pl.ds(off, ECH)])
        return carry

    lax.fori_loop(0, EW2 // ECH, chunk, 0)


@functools.lru_cache(maxsize=None)
def _make_edge():
    mesh = plsc.VectorSubcoreMesh(core_axis_name="c", subcore_axis_name="s")
    return pl.kernel(
        _edge_body,
        out_type=jax.ShapeDtypeStruct((2, E), jnp.float32),
        mesh=mesh,
        scratch_types=[
            pltpu.VMEM((2, N), jnp.float32),
            pltpu.VMEM((ECH,), jnp.int32),
            pltpu.VMEM((ECH,), jnp.int32),
            pltpu.VMEM((ECH,), jnp.float32),
        ],
    )


def _dot(a, b):
    return jnp.dot(a, b, preferred_element_type=jnp.float32)


def _m0_body(x_ref, w_ref, o_ref):
    o_ref[...] = _dot(x_ref[...], w_ref[...])


def _tc_m0(x, w02):
    return pl.pallas_call(
        _m0_body,
        out_shape=jax.ShapeDtypeStruct((N, DP1), jnp.float32),
    )(x, w02)


def _gru_math(p_ref, h_ref, wi_r, wi_z, wi_n, wh_r, wh_z, wh_n,
              bi_r, bi_z, bi_n, bh_r, bh_z, bh_n):
    agg = p_ref[0] + p_ref[1]
    h = h_ref[...]
    r = jax.nn.sigmoid(_dot(agg, wi_r[...]) + bi_r[...]
                       + _dot(h, wh_r[...]) + bh_r[...])
    z = jax.nn.sigmoid(_dot(agg, wi_z[...]) + bi_z[...]
                       + _dot(h, wh_z[...]) + bh_z[...])
    n = jnp.tanh(_dot(agg, wi_n[...]) + bi_n[...]
                 + r * (_dot(h, wh_n[...]) + bh_n[...]))
    return n + z * (h - n)


def _gru_mid_body(p_ref, h_ref, wi_r, wi_z, wi_n, wh_r, wh_z, wh_n,
                  bi_r, bi_z, bi_n, bh_r, bh_z, bh_n, wnext,
                  h_out, m_out):
    hn = _gru_math(p_ref, h_ref, wi_r, wi_z, wi_n, wh_r, wh_z, wh_n,
                   bi_r, bi_z, bi_n, bh_r, bh_z, bh_n)
    h_out[...] = hn
    m_out[...] = _dot(hn, wnext[...])


def _gru_bridge_body(p_ref, h_ref, wi_r, wi_z, wi_n, wh_r, wh_z, wh_n,
                     bi_r, bi_z, bi_n, bh_r, bh_z, bh_n, wnext,
                     h_out, m_out):
    hn = _gru_math(p_ref, h_ref, wi_r, wi_z, wi_n, wh_r, wh_z, wh_n,
                   bi_r, bi_z, bi_n, bh_r, bh_z, bh_n)
    hr = jax.nn.relu(hn)
    h_out[...] = jnp.concatenate([hr, jnp.zeros_like(hr)], axis=1)
    m_out[...] = _dot(hr, wnext[...])


def _gru_final_body(p_ref, h_ref, wi_r, wi_z, wi_n, wh_r, wh_z, wh_n,
                    bi_r, bi_z, bi_n, bh_r, bh_z, bh_n,
                    wn_t, bn_v, wcat, bcat, tbl_out):
    hn = _gru_math(p_ref, h_ref, wi_r, wi_z, wi_n, wh_r, wh_z, wh_n,
                   bi_r, bi_z, bi_n, bh_r, bh_z, bh_n)
    hr = jax.nn.relu(hn)
    hh = jax.nn.relu(_dot(hr, wn_t[...]) + bn_v[...])
    tbl_out[...] = _dot(hh, wcat[...]) + bcat[...]


def _tc_gru(body, parts, h, extra, out_shapes):
    return pl.pallas_call(
        body,
        out_shape=out_shapes,
    )(parts, h, *extra)


def _split_gru_weights(Wih, Whh, bih, bhh, D, DPin):
    out = []
    for g in range(3):
        w = Wih[g * D:(g + 1) * D].T
        out.append(jnp.pad(w, ((0, DPin - D), (0, 0))))
    for g in range(3):
        out.append(Whh[g * D:(g + 1) * D].T)
    for g in range(3):
        out.append(bih[g * D:(g + 1) * D])
    for g in range(3):
        out.append(bhh[g * D:(g + 1) * D])
    return out


def kernel(node_features, edge_index, distance, W1, Wih1, Whh1, bih1, bhh1,
           W2, Wih2, Whh2, bih2, bhh2, Wn, bn, We, be):
    src = edge_index[0]
    dst = edge_index[1]
    zeros1 = jnp.zeros((N, DP1), jnp.float32)
    zeros2 = jnp.zeros((N, DP2), jnp.float32)

    w02 = jnp.pad(W1[0][:2], ((0, 0), (0, DP1 - D1)))
    w1_1 = jnp.pad(W1[1], ((0, 0), (0, DP1 - D1)))
    w2_0 = jnp.pad(W2[0][:D1], ((0, 0), (0, DP2 - D2)))
    w2_1 = jnp.pad(W2[1], ((0, 0), (0, DP2 - D2)))
    w2_2 = jnp.pad(W2[2], ((0, 0), (0, DP2 - D2)))
    gw1 = _split_gru_weights(Wih1, Whh1, bih1, bhh1, D1, DP1)
    gw2 = _split_gru_weights(Wih2, Whh2, bih2, bhh2, D2, DP2)
    wn_t = Wn.T
    wcat = jnp.concatenate([We[:, :D2].T, We[:, D2:].T], axis=1)
    bcat = jnp.concatenate([jnp.zeros((2,), jnp.float32), be])

    h0 = jnp.pad(node_features, ((0, 0), (0, D1 - 2)))

    msg1 = _make_msg(DP1)
    msg2 = _make_msg(DP2)

    m = _tc_m0(node_features, w02)
    p = msg1(m, src, dst, distance, zeros1)
    h1, m = _tc_gru(_gru_mid_body, p, h0, gw1 + [w1_1],
                    [jax.ShapeDtypeStruct((N, D1), jnp.float32),
                     jax.ShapeDtypeStruct((N, DP1), jnp.float32)])
    p = msg1(m, src, dst, distance, zeros1)
    hb, m = _tc_gru(_gru_bridge_body, p, h1, gw1 + [w2_0],
                    [jax.ShapeDtypeStruct((N, D2), jnp.float32),
                     jax.ShapeDtypeStruct((N, DP2), jnp.float32)])

    p = msg2(m, src, dst, distance, zeros2)
    h3, m = _tc_gru(_gru_mid_body, p, hb, gw2 + [w2_1],
                    [jax.ShapeDtypeStruct((N, D2), jnp.float32),
                     jax.ShapeDtypeStruct((N, DP2), jnp.float32)])
    p = msg2(m, src, dst, distance, zeros2)
    h4, m = _tc_gru(_gru_mid_body, p, h3, gw2 + [w2_2],
                    [jax.ShapeDtypeStruct((N, D2), jnp.float32),
                     jax.ShapeDtypeStruct((N, DP2), jnp.float32)])
    p = msg2(m, src, dst, distance, zeros2)
    tbl_n4 = _tc_gru(
        _gru_final_body, p, h4, gw2 + [wn_t, bn, wcat, bcat],
        jax.ShapeDtypeStruct((N, 4), jnp.float32))

    tbl = tbl_n4.T

    out2 = _make_edge()(tbl, src, dst)
    return out2.T


# speedup vs baseline: 6.6552x; 6.6552x over previous
import functools

import jax
import jax.numpy as jnp
from jax import lax
from jax.experimental import pallas as pl
from jax.experimental.pallas import tpu as pltpu
from jax.experimental.pallas import tpu_sc as plsc

N = 10000
E = 640000
NC = 2
NS = 16
LANES = 16
NW = NC * NS
EW = E // NW
CH = 80
NCH = EW // CH
NP = 10240
RPS = NP // NS

D1, DP1 = 50, 64
D2, DP2 = 100, 112

_GDN = lax.GatherDimensionNumbers(
    offset_dims=(), collapsed_slice_dims=(0,), start_index_map=(0,))


def _bcast_lane(vec, l):
    idx = jnp.full((LANES, 1), l, jnp.int32)
    return lax.gather(vec, idx, _GDN, (1,),
                      mode=lax.GatherScatterMode.PROMISE_IN_BOUNDS)


def _msg_body(DP, m_hbm, src_hbm, dst_hbm, dist_hbm, zeros_hbm, out_hbm,
              idx_s, idx_d, dist_v, rows_v, agg_sh, sem):
    cid = lax.axis_index("c")
    sid = lax.axis_index("s")
    wid = cid * NS + sid
    base = wid * EW
    r0 = sid * RPS
    pltpu.sync_copy(zeros_hbm.at[pl.ds(r0, RPS)], agg_sh.at[pl.ds(r0, RPS)])
    plsc.subcore_barrier()

    def chunk(c, carry):
        off = base + c * CH
        pltpu.sync_copy(src_hbm.at[pl.ds(off, CH)], idx_s)
        pltpu.sync_copy(dst_hbm.at[pl.ds(off, CH)], idx_d)
        pltpu.sync_copy(dist_hbm.at[pl.ds(off, CH)], dist_v)
        pltpu.async_copy(m_hbm.at[idx_s], rows_v, sem).wait()
        for g in range(CH // LANES):
            dvec = dist_v[pl.ds(g * LANES, LANES)]
            for l in range(LANES):
                dsp = _bcast_lane(dvec, l)
                e = g * LANES + l
                for j in range(DP // LANES):
                    sl = pl.ds(j * LANES, LANES)
                    rows_v[e, sl] = rows_v[e, sl] * dsp
        pltpu.sync_copy(rows_v, agg_sh.at[idx_d], add=True)
        return carry

    lax.fori_loop(0, NCH, chunk, 0)
    plsc.subcore_barrier()
    pltpu.sync_copy(agg_sh.at[pl.ds(r0, RPS)],
                    out_hbm.at[cid, pl.ds(r0, RPS)])


@functools.lru_cache(maxsize=None)
def _make_msg(DP):
    mesh = plsc.VectorSubcoreMesh(core_axis_name="c", subcore_axis_name="s")
    return pl.kernel(
        functools.partial(_msg_body, DP),
        out_type=jax.ShapeDtypeStruct((NC, NP, DP), jnp.float32),
        mesh=mesh,
        compiler_params=pltpu.CompilerParams(use_tc_tiling_on_sc=False),
        scratch_types=[
            pltpu.VMEM((CH,), jnp.int32),
            pltpu.VMEM((CH,), jnp.int32),
            pltpu.VMEM((CH,), jnp.float32),
            pltpu.VMEM((CH, DP), jnp.float32),
            pltpu.VMEM_SHARED((NP, DP), jnp.float32),
            pltpu.SemaphoreType.DMA,
        ],
    )


EW2 = E // 16
ECH = 800


def _edge_body(tbl_hbm, src_hbm, dst_hbm, out_hbm, tb_s, tb_d, sidx, didx, out_v):
    cid = lax.axis_index("c")
    sid = lax.axis_index("s")
    wid = cid * NS + sid
    comp = wid // 16
    part = wid % 16
    base = part * EW2
    pltpu.sync_copy(tbl_hbm.at[pl.ds(comp * N, N)], tb_s)
    pltpu.sync_copy(tbl_hbm.at[pl.ds((comp + 2) * N, N)], tb_d)

    def chunk(c, carry):
        off = base + c * ECH
        pltpu.sync_copy(src_hbm.at[pl.ds(off, ECH)], sidx)
        pltpu.sync_copy(dst_hbm.at[pl.ds(off, ECH)], didx)
        for g in range(ECH // LANES):
            sl = pl.ds(g * LANES, LANES)
            sv = sidx[sl]
            dv = didx[sl]
            s = (plsc.load_gather(tb_s, [sv])
                 + plsc.load_gather(tb_d, [dv]))
            out_v[sl] = s
        pltpu.sync_copy(out_v, out_hbm.at[comp, pl.ds(off, ECH)])
        return carry

    lax.fori_loop(0, EW2 // ECH, chunk, 0)


@functools.lru_cache(maxsize=None)
def _make_edge():
    mesh = plsc.VectorSubcoreMesh(core_axis_name="c", subcore_axis_name="s")
    return pl.kernel(
        _edge_body,
        out_type=jax.ShapeDtypeStruct((2, E), jnp.float32),
        mesh=mesh,
        compiler_params=pltpu.CompilerParams(use_tc_tiling_on_sc=False,
                                             needs_layout_passes=False),
        scratch_types=[
            pltpu.VMEM((N,), jnp.float32),
            pltpu.VMEM((N,), jnp.float32),
            pltpu.VMEM((ECH,), jnp.int32),
            pltpu.VMEM((ECH,), jnp.int32),
            pltpu.VMEM((ECH,), jnp.float32),
        ],
    )


def _dot(a, b):
    return jnp.dot(a, b, preferred_element_type=jnp.float32)


def _m0_body(x_ref, w_ref, o_ref):
    o_ref[:N] = _dot(x_ref[...], w_ref[...])


def _tc_m0(x, w02):
    return pl.pallas_call(
        _m0_body,
        out_shape=jax.ShapeDtypeStruct((NP, DP1), jnp.float32),
    )(x, w02)


def _gru_math(p_ref, h_ref, wi_r, wi_z, wi_n, wh_r, wh_z, wh_n,
              bi_r, bi_z, bi_n, bh_r, bh_z, bh_n):
    agg = p_ref[0, :N] + p_ref[1, :N]
    h = h_ref[...]
    r = jax.nn.sigmoid(_dot(agg, wi_r[...]) + bi_r[...]
                       + _dot(h, wh_r[...]) + bh_r[...])
    z = jax.nn.sigmoid(_dot(agg, wi_z[...]) + bi_z[...]
                       + _dot(h, wh_z[...]) + bh_z[...])
    n = jnp.tanh(_dot(agg, wi_n[...]) + bi_n[...]
                 + r * (_dot(h, wh_n[...]) + bh_n[...]))
    return n + z * (h - n)


def _gru_mid_body(p_ref, h_ref, wi_r, wi_z, wi_n, wh_r, wh_z, wh_n,
                  bi_r, bi_z, bi_n, bh_r, bh_z, bh_n, wnext,
                  h_out, m_out):
    hn = _gru_math(p_ref, h_ref, wi_r, wi_z, wi_n, wh_r, wh_z, wh_n,
                   bi_r, bi_z, bi_n, bh_r, bh_z, bh_n)
    h_out[...] = hn
    m_out[:N] = _dot(hn, wnext[...])


def _gru_bridge_body(p_ref, h_ref, wi_r, wi_z, wi_n, wh_r, wh_z, wh_n,
                     bi_r, bi_z, bi_n, bh_r, bh_z, bh_n, wnext,
                     h_out, m_out):
    hn = _gru_math(p_ref, h_ref, wi_r, wi_z, wi_n, wh_r, wh_z, wh_n,
                   bi_r, bi_z, bi_n, bh_r, bh_z, bh_n)
    hr = jax.nn.relu(hn)
    h_out[...] = jnp.concatenate([hr, jnp.zeros_like(hr)], axis=1)
    m_out[:N] = _dot(hr, wnext[...])


def _gru_final_body(p_ref, h_ref, wi_r, wi_z, wi_n, wh_r, wh_z, wh_n,
                    bi_r, bi_z, bi_n, bh_r, bh_z, bh_n,
                    wn_t, bn_v, wcat, bcat, tbl_out):
    hn = _gru_math(p_ref, h_ref, wi_r, wi_z, wi_n, wh_r, wh_z, wh_n,
                   bi_r, bi_z, bi_n, bh_r, bh_z, bh_n)
    hr = jax.nn.relu(hn)
    hh = jax.nn.relu(_dot(hr, wn_t[...]) + bn_v[...])
    tbl_out[...] = _dot(hh, wcat[...]) + bcat[...]


def _tc_gru(body, parts, h, extra, out_shapes):
    return pl.pallas_call(
        body,
        out_shape=out_shapes,
    )(parts, h, *extra)


def _split_gru_weights(Wih, Whh, bih, bhh, D, DPin):
    out = []
    for g in range(3):
        w = Wih[g * D:(g + 1) * D].T
        out.append(jnp.pad(w, ((0, DPin - D), (0, 0))))
    for g in range(3):
        out.append(Whh[g * D:(g + 1) * D].T)
    for g in range(3):
        out.append(bih[g * D:(g + 1) * D])
    for g in range(3):
        out.append(bhh[g * D:(g + 1) * D])
    return out


def kernel(node_features, edge_index, distance, W1, Wih1, Whh1, bih1, bhh1,
           W2, Wih2, Whh2, bih2, bhh2, Wn, bn, We, be):
    src = edge_index[0]
    dst = edge_index[1]
    zeros1 = jnp.zeros((NP, DP1), jnp.float32)
    zeros2 = jnp.zeros((NP, DP2), jnp.float32)

    w02 = jnp.pad(W1[0][:2], ((0, 0), (0, DP1 - D1)))
    w1_1 = jnp.pad(W1[1], ((0, 0), (0, DP1 - D1)))
    w2_0 = jnp.pad(W2[0][:D1], ((0, 0), (0, DP2 - D2)))
    w2_1 = jnp.pad(W2[1], ((0, 0), (0, DP2 - D2)))
    w2_2 = jnp.pad(W2[2], ((0, 0), (0, DP2 - D2)))
    gw1 = _split_gru_weights(Wih1, Whh1, bih1, bhh1, D1, DP1)
    gw2 = _split_gru_weights(Wih2, Whh2, bih2, bhh2, D2, DP2)
    wn_t = Wn.T
    wcat = jnp.concatenate([We[:, :D2].T, We[:, D2:].T], axis=1)
    bcat = jnp.concatenate([jnp.zeros((2,), jnp.float32), be])

    h0 = jnp.pad(node_features, ((0, 0), (0, D1 - 2)))

    msg1 = _make_msg(DP1)
    msg2 = _make_msg(DP2)

    m = _tc_m0(node_features, w02)
    p = msg1(m, src, dst, distance, zeros1)
    h1, m = _tc_gru(_gru_mid_body, p, h0, gw1 + [w1_1],
                    [jax.ShapeDtypeStruct((N, D1), jnp.float32),
                     jax.ShapeDtypeStruct((NP, DP1), jnp.float32)])
    p = msg1(m, src, dst, distance, zeros1)
    hb, m = _tc_gru(_gru_bridge_body, p, h1, gw1 + [w2_0],
                    [jax.ShapeDtypeStruct((N, D2), jnp.float32),
                     jax.ShapeDtypeStruct((NP, DP2), jnp.float32)])

    p = msg2(m, src, dst, distance, zeros2)
    h3, m = _tc_gru(_gru_mid_body, p, hb, gw2 + [w2_1],
                    [jax.ShapeDtypeStruct((N, D2), jnp.float32),
                     jax.ShapeDtypeStruct((NP, DP2), jnp.float32)])
    p = msg2(m, src, dst, distance, zeros2)
    h4, m = _tc_gru(_gru_mid_body, p, h3, gw2 + [w2_2],
                    [jax.ShapeDtypeStruct((N, D2), jnp.float32),
                     jax.ShapeDtypeStruct((NP, DP2), jnp.float32)])
    p = msg2(m, src, dst, distance, zeros2)
    tbl_n4 = _tc_gru(
        _gru_final_body, p, h4, gw2 + [wn_t, bn, wcat, bcat],
        jax.ShapeDtypeStruct((N, 4), jnp.float32))

    tblf = tbl_n4.T.reshape(4 * N)

    out2 = _make_edge()(tblf, src, dst)
    return out2.T
